# Initial kernel scaffold; baseline (speedup 1.0000x reference)
import jax
import jax.numpy as jnp
from jax.experimental import pallas as pl


def kernel(params, x, edge_index, edge_attr, batch):
    raise NotImplementedError("write your pallas kernel here")


# speedup vs baseline: 2.7668x; 2.7668x over previous
import functools

import jax
import jax.numpy as jnp
from jax import lax
from jax.experimental import pallas as pl
from jax.experimental.pallas import tpu as pltpu
from jax.experimental.pallas import tpu_sc as plsc

HID = 256
H2 = 128
NL = 5
NF = 9
EF = 3
NVOC = 64
EVOC = 16
NG = 64
NN = 10000
NE = 160000
TBL = EVOC ** EF
NC, NS = 2, 16
BLK = 1000
NB = NN // BLK
K = 80
EPS_N = NE // NS
NCH = EPS_N // K
NNP = 10240
RPS = NNP // NS
F32 = jnp.float32


def _tables_body(es_ref, w_ref, b_ref, out_ref):
    r = lax.broadcasted_iota(jnp.int32, (TBL, EVOC), 0)
    i = lax.broadcasted_iota(jnp.int32, (TBL, EVOC), 1)
    a0 = ((r // 256) == i).astype(F32)
    a1 = (((r // 16) % 16) == i).astype(F32)
    a2 = ((r % 16) == i).astype(F32)
    for l in range(NL):
        w = w_ref[l]
        t = jnp.dot(a0, jnp.dot(es_ref[0], w, preferred_element_type=F32),
                    preferred_element_type=F32)
        t += jnp.dot(a1, jnp.dot(es_ref[1], w, preferred_element_type=F32),
                     preferred_element_type=F32)
        t += jnp.dot(a2, jnp.dot(es_ref[2], w, preferred_element_type=F32),
                     preferred_element_type=F32)
        t += b_ref[l]
        out_ref[l, :TBL, :] = t[:, :H2]
        out_ref[l, TBL:, :] = t[:, H2:]


def _tables_call(estack, wstack, bstack):
    return pl.pallas_call(
        _tables_body,
        out_shape=jax.ShapeDtypeStruct((NL, 2 * TBL, H2), F32),
    )(estack, wstack, bstack)


def _embed_body(x_ref, nt_ref, g_ref, b_ref, h_ref, hc_ref):
    xb = x_ref[...]
    iota = lax.broadcasted_iota(jnp.int32, (BLK, NVOC), 1)
    acc = jnp.zeros((BLK, HID), F32)
    for f in range(NF):
        oh = (xb[:, f:f + 1] == iota).astype(F32)
        acc += jnp.dot(oh, nt_ref[f], preferred_element_type=F32)
    mu = jnp.mean(acc, axis=1, keepdims=True)
    var = jnp.mean((acc - mu) ** 2, axis=1, keepdims=True)
    hn = (acc - mu) * lax.rsqrt(var + 1e-5) * g_ref[...] + b_ref[...]
    h_ref[...] = hn
    hc_ref[0] = hn[:, :H2]
    hc_ref[1] = hn[:, H2:]


def _embed_call(x, ntab, ln_g, ln_b):
    return pl.pallas_call(
        _embed_body,
        grid=(NB,),
        in_specs=[
            pl.BlockSpec((BLK, NF), lambda i: (i, 0)),
            pl.BlockSpec((NF, NVOC, HID), lambda i: (0, 0, 0)),
            pl.BlockSpec((1, HID), lambda i: (0, 0)),
            pl.BlockSpec((1, HID), lambda i: (0, 0)),
        ],
        out_specs=[
            pl.BlockSpec((BLK, HID), lambda i: (i, 0)),
            pl.BlockSpec((2, BLK, H2), lambda i: (0, i, 0)),
        ],
        out_shape=[
            jax.ShapeDtypeStruct((NN, HID), F32),
            jax.ShapeDtypeStruct((2, NN, H2), F32),
        ],
    )(x, ntab, ln_g, ln_b)


def _mp_body(hcat, tcat, src2, eidx2, dst, zeros, out,
             sidx, eidxv, didx, hbuf, tbuf, aggr, hsem, tsem):
    c = lax.axis_index("c")
    s = lax.axis_index("s")
    pltpu.sync_copy(zeros, aggr.at[pl.ds(s * RPS, RPS)])
    plsc.subcore_barrier()

    def chunk(i, carry):
        base = s * EPS_N + i * K
        cbase = c * NE + base
        pltpu.sync_copy(src2.at[pl.ds(cbase, K)], sidx)
        pltpu.sync_copy(eidx2.at[pl.ds(cbase, K)], eidxv)
        pltpu.sync_copy(dst.at[pl.ds(base, K)], didx)
        cp1 = pltpu.async_copy(hcat.at[sidx], hbuf, hsem)
        cp2 = pltpu.async_copy(tcat.at[eidxv], tbuf, tsem)
        cp1.wait()
        cp2.wait()

        def row(r, c2):
            for q in range(H2 // 16):
                sl = pl.ds(q * 16, 16)
                hbuf[r, sl] = jnp.maximum(hbuf[r, sl] + tbuf[r, sl], 0.0)
            return c2

        lax.fori_loop(0, K, row, 0)
        pltpu.sync_copy(hbuf, aggr.at[didx], add=True)
        return carry

    lax.fori_loop(0, NCH, chunk, 0)
    plsc.subcore_barrier()
    pltpu.sync_copy(aggr.at[pl.ds(s * RPS, RPS)],
                    out.at[pl.ds(c * NNP + s * RPS, RPS)])


@functools.cache
def _mp_kernel():
    return pl.kernel(
        _mp_body,
        out_type=jax.ShapeDtypeStruct((2 * NNP, H2), F32),
        mesh=plsc.VectorSubcoreMesh(core_axis_name="c", subcore_axis_name="s",
                                    num_cores=NC, num_subcores=NS),
        scratch_types=[
            pltpu.VMEM((K,), jnp.int32),
            pltpu.VMEM((K,), jnp.int32),
            pltpu.VMEM((K,), jnp.int32),
            pltpu.VMEM((K, H2), F32),
            pltpu.VMEM((K, H2), F32),
            pltpu.VMEM_SHARED((NNP, H2), F32),
            pltpu.SemaphoreType.DMA,
            pltpu.SemaphoreType.DMA,
        ],
    )


def _mp_call(*args):
    return _mp_kernel()(*args)


def _mlp1_body(eps_ref, h_ref, a2_ref, w1_ref, b1_ref, w2_ref, b2_ref,
               z2_ref, ps_ref):
    aggr = jnp.concatenate([a2_ref[0], a2_ref[1]], axis=1)
    z = (1.0 + eps_ref[0, 0]) * h_ref[...] + aggr
    hid = jnp.maximum(
        jnp.dot(z, w1_ref[...], preferred_element_type=F32) + b1_ref[...], 0.0)
    z2 = jnp.dot(hid, w2_ref[...], preferred_element_type=F32) + b2_ref[...]
    z2_ref[...] = z2
    ps_ref[0, 0:1, :] = jnp.sum(z2, axis=0, keepdims=True)
    ps_ref[0, 1:2, :] = jnp.sum(z2 * z2, axis=0, keepdims=True)


def _mlp1_call(eps, h, a3, w1, b1, w2, b2):
    return pl.pallas_call(
        _mlp1_body,
        grid=(NB,),
        in_specs=[
            pl.BlockSpec((1, 1), lambda i: (0, 0)),
            pl.BlockSpec((BLK, HID), lambda i: (i, 0)),
            pl.BlockSpec((2, BLK, H2), lambda i: (0, i, 0)),
            pl.BlockSpec((HID, HID), lambda i: (0, 0)),
            pl.BlockSpec((1, HID), lambda i: (0, 0)),
            pl.BlockSpec((HID, HID), lambda i: (0, 0)),
            pl.BlockSpec((1, HID), lambda i: (0, 0)),
        ],
        out_specs=[
            pl.BlockSpec((BLK, HID), lambda i: (i, 0)),
            pl.BlockSpec((1, 2, HID), lambda i: (i, 0, 0)),
        ],
        out_shape=[
            jax.ShapeDtypeStruct((NN, HID), F32),
            jax.ShapeDtypeStruct((NB, 2, HID), F32),
        ],
    )(eps, h, a3, w1, b1, w2, b2)


def _mlp2_body(h_ref, z2_ref, ps_ref, g_ref, b_ref, ho_ref, hc_ref):
    ps = ps_ref[...]
    s1 = jnp.sum(ps[:, 0:1, :], axis=0)
    s2 = jnp.sum(ps[:, 1:2, :], axis=0)
    mean = s1 / NN
    var = s2 / NN - mean * mean
    z = (z2_ref[...] - mean) * lax.rsqrt(var + 1e-5) * g_ref[...] + b_ref[...]
    z = jnp.maximum(z, 0.0)
    hn = h_ref[...] + z
    ho_ref[...] = hn
    hc_ref[0] = hn[:, :H2]
    hc_ref[1] = hn[:, H2:]


def _mlp2_call(h, z2, ps, bn_g, bn_b):
    return pl.pallas_call(
        _mlp2_body,
        grid=(NB,),
        in_specs=[
            pl.BlockSpec((BLK, HID), lambda i: (i, 0)),
            pl.BlockSpec((BLK, HID), lambda i: (i, 0)),
            pl.BlockSpec((NB, 2, HID), lambda i: (0, 0, 0)),
            pl.BlockSpec((1, HID), lambda i: (0, 0)),
            pl.BlockSpec((1, HID), lambda i: (0, 0)),
        ],
        out_specs=[
            pl.BlockSpec((BLK, HID), lambda i: (i, 0)),
            pl.BlockSpec((2, BLK, H2), lambda i: (0, i, 0)),
        ],
        out_shape=[
            jax.ShapeDtypeStruct((NN, HID), F32),
            jax.ShapeDtypeStruct((2, NN, H2), F32),
        ],
    )(h, z2, ps, bn_g, bn_b)


def _readout_body(h_ref, bt_ref, w1_ref, b1_ref, w2_ref, b2_ref, out_ref):
    h = h_ref[...]
    g1 = jnp.maximum(
        jnp.dot(h, w1_ref[...], preferred_element_type=F32) + b1_ref[...], 0.0)
    gate = jnp.dot(g1, w2_ref[...], preferred_element_type=F32) + b2_ref[...]
    oh = bt_ref[...] == lax.broadcasted_iota(jnp.int32, (NN, NG), 1)
    ohf = oh.astype(F32)
    gmax = jnp.max(jnp.where(oh, gate, -3e38), axis=0, keepdims=True)
    gm_n = jnp.sum(jnp.where(oh, gmax, 0.0), axis=1, keepdims=True)
    gexp = jnp.exp(gate - gm_n)
    denom = jnp.sum(ohf * gexp, axis=0, keepdims=True)
    den_n = jnp.sum(jnp.where(oh, denom, 0.0), axis=1, keepdims=True)
    alpha = gexp / den_n
    out_ref[...] = lax.dot_general(ohf, alpha * h, (((0,), (0,)), ((), ())),
                                   preferred_element_type=F32)


def _readout_call(h, bt, w1, b1, w2, b2):
    return pl.pallas_call(
        _readout_body,
        out_shape=jax.ShapeDtypeStruct((NG, HID), F32),
    )(h, bt, w1, b1, w2, b2)


def kernel(params, x, edge_index, edge_attr, batch):
    p = params
    estack = jnp.stack(p["edge_embs"])
    wstack = jnp.stack([lp["linW"] for lp in p["layers"]])
    bstack = jnp.stack([lp["linb"] for lp in p["layers"]])[:, None, :]
    tcat = _tables_call(estack, wstack, bstack)

    ntab = jnp.stack(p["node_embs"])
    h, hcat = _embed_call(x.astype(jnp.int32), ntab,
                          p["ln_g"][None], p["ln_b"][None])

    src = edge_index[0].astype(jnp.int32)
    dst = edge_index[1].astype(jnp.int32)
    ea = edge_attr.astype(jnp.int32)
    eidx = ea[:, 0] * (EVOC * EVOC) + ea[:, 1] * EVOC + ea[:, 2]
    src2 = jnp.concatenate([src, src + NN])
    eidx2 = jnp.concatenate([eidx, eidx + TBL])
    zeros = jnp.zeros((RPS, H2), F32)

    for l in range(NL):
        lp = p["layers"][l]
        aggr2 = _mp_call(hcat.reshape(2 * NN, H2), tcat[l], src2, eidx2,
                         dst, zeros)
        a3 = aggr2.reshape(2, NNP, H2)[:, :NN, :]
        z2, ps = _mlp1_call(lp["eps"].reshape(1, 1).astype(F32), h, a3,
                            lp["W1"], lp["b1"][None], lp["W2"], lp["b2"][None])
        h, hcat = _mlp2_call(h, z2, ps, lp["bn_g"][None], lp["bn_b"][None])

    graph_emb = _readout_call(h, batch.astype(jnp.int32)[:, None],
                              p["gW1"], p["gb1"][None], p["gW2"],
                              p["gb2"].reshape(1, 1))
    return graph_emb, h


# speedup vs baseline: 3.9750x; 1.4367x over previous
import functools

import jax
import jax.numpy as jnp
from jax import lax
from jax.experimental import pallas as pl
from jax.experimental.pallas import tpu as pltpu
from jax.experimental.pallas import tpu_sc as plsc

HID = 256
H2 = 128
NL = 5
NF = 9
EF = 3
NVOC = 64
EVOC = 16
NG = 64
NN = 10000
NE = 160000
TBL = EVOC ** EF
NC, NS = 2, 16
BLK = 1000
NB = NN // BLK
K = 80
EPS_N = NE // NS
NCH = EPS_N // K
NNP = 10240
RPS = NNP // NS
F32 = jnp.float32


def _tables_body(es_ref, w_ref, b_ref, out_ref):
    r = lax.broadcasted_iota(jnp.int32, (TBL, EVOC), 0)
    i = lax.broadcasted_iota(jnp.int32, (TBL, EVOC), 1)
    a0 = ((r // 256) == i).astype(F32)
    a1 = (((r // 16) % 16) == i).astype(F32)
    a2 = ((r % 16) == i).astype(F32)
    for l in range(NL):
        w = w_ref[l]
        t = jnp.dot(a0, jnp.dot(es_ref[0], w, preferred_element_type=F32),
                    preferred_element_type=F32)
        t += jnp.dot(a1, jnp.dot(es_ref[1], w, preferred_element_type=F32),
                     preferred_element_type=F32)
        t += jnp.dot(a2, jnp.dot(es_ref[2], w, preferred_element_type=F32),
                     preferred_element_type=F32)
        t += b_ref[l]
        out_ref[l, :TBL, :] = t[:, :H2]
        out_ref[l, TBL:, :] = t[:, H2:]


def _tables_call(estack, wstack, bstack):
    return pl.pallas_call(
        _tables_body,
        out_shape=jax.ShapeDtypeStruct((NL, 2 * TBL, H2), F32),
    )(estack, wstack, bstack)


def _embed_body(x_ref, nt_ref, g_ref, b_ref, h_ref, hc_ref):
    xb = x_ref[...]
    iota = lax.broadcasted_iota(jnp.int32, (BLK, NVOC), 1)
    acc = jnp.zeros((BLK, HID), F32)
    for f in range(NF):
        oh = (xb[:, f:f + 1] == iota).astype(F32)
        acc += jnp.dot(oh, nt_ref[f], preferred_element_type=F32)
    mu = jnp.mean(acc, axis=1, keepdims=True)
    var = jnp.mean((acc - mu) ** 2, axis=1, keepdims=True)
    hn = (acc - mu) * lax.rsqrt(var + 1e-5) * g_ref[...] + b_ref[...]
    h_ref[...] = hn
    hc_ref[0] = hn[:, :H2]
    hc_ref[1] = hn[:, H2:]


def _embed_call(x, ntab, ln_g, ln_b):
    return pl.pallas_call(
        _embed_body,
        grid=(NB,),
        in_specs=[
            pl.BlockSpec((BLK, NF), lambda i: (i, 0)),
            pl.BlockSpec((NF, NVOC, HID), lambda i: (0, 0, 0)),
            pl.BlockSpec((1, HID), lambda i: (0, 0)),
            pl.BlockSpec((1, HID), lambda i: (0, 0)),
        ],
        out_specs=[
            pl.BlockSpec((BLK, HID), lambda i: (i, 0)),
            pl.BlockSpec((2, BLK, H2), lambda i: (0, i, 0)),
        ],
        out_shape=[
            jax.ShapeDtypeStruct((NN, HID), F32),
            jax.ShapeDtypeStruct((2, NN, H2), F32),
        ],
    )(x, ntab, ln_g, ln_b)


def _mp_body(hcat, tcat, src2, eidx2, dst, zeros, out,
             sidx0, eidxv0, didx0, hbuf0, tbuf0,
             sidx1, eidxv1, didx1, hbuf1, tbuf1,
             aggr, gsem0, gsem1, ssem0, ssem1):
    c = lax.axis_index("c")
    s = lax.axis_index("s")
    pltpu.sync_copy(zeros, aggr.at[pl.ds(s * RPS, RPS)])
    plsc.subcore_barrier()

    slots = ((sidx0, eidxv0, didx0, hbuf0, tbuf0, gsem0, ssem0),
             (sidx1, eidxv1, didx1, hbuf1, tbuf1, gsem1, ssem1))

    def load_fire(i, slot):
        si, ei, di, hb, tb, gs, _ = slot
        base = s * EPS_N + i * K
        cbase = c * NE + base
        pltpu.sync_copy(src2.at[pl.ds(cbase, K)], si)
        pltpu.sync_copy(eidx2.at[pl.ds(cbase, K)], ei)
        pltpu.sync_copy(dst.at[pl.ds(base, K)], di)
        pltpu.async_copy(hcat.at[si], hb, gs)
        pltpu.async_copy(tcat.at[ei], tb, gs)

    def drain_g(slot):
        _, _, _, hb, tb, gs, _ = slot
        pltpu.make_async_copy(hcat.at[pl.ds(0, K)], hb, gs).wait()
        pltpu.make_async_copy(tcat.at[pl.ds(0, K)], tb, gs).wait()

    def compute(slot):
        _, _, _, hb, tb, _, _ = slot

        @plsc.parallel_loop(0, K, step=1, unroll=4)
        def _(r):
            for q in range(H2 // 16):
                sl = pl.ds(q * 16, 16)
                hb[r, sl] = jnp.maximum(hb[r, sl] + tb[r, sl], 0.0)

    def fire_scatter(slot):
        _, _, di, hb, _, _, ss = slot
        pltpu.async_copy(hb, aggr.at[di], ss, add=True)

    def drain_s(slot):
        _, _, _, hb, _, _, ss = slot
        pltpu.make_async_copy(hcat.at[pl.ds(0, K)], hb, ss).wait()

    load_fire(0, slots[0])
    load_fire(1, slots[1])
    drain_g(slots[0])
    compute(slots[0])
    fire_scatter(slots[0])

    def pair(p, carry):
        i1 = 2 * p + 1
        drain_s(slots[0])
        load_fire(i1 + 1, slots[0])
        drain_g(slots[1])
        compute(slots[1])
        fire_scatter(slots[1])
        drain_s(slots[1])
        load_fire(i1 + 2, slots[1])
        drain_g(slots[0])
        compute(slots[0])
        fire_scatter(slots[0])
        return carry

    lax.fori_loop(0, (NCH - 3) // 2, pair, 0)

    drain_s(slots[0])
    load_fire(NCH - 1, slots[0])
    drain_g(slots[1])
    compute(slots[1])
    fire_scatter(slots[1])
    drain_g(slots[0])
    compute(slots[0])
    fire_scatter(slots[0])
    drain_s(slots[1])
    drain_s(slots[0])

    plsc.subcore_barrier()
    pltpu.sync_copy(aggr.at[pl.ds(s * RPS, RPS)],
                    out.at[pl.ds(c * NNP + s * RPS, RPS)])


@functools.cache
def _mp_kernel():
    return pl.kernel(
        _mp_body,
        out_type=jax.ShapeDtypeStruct((2 * NNP, H2), F32),
        mesh=plsc.VectorSubcoreMesh(core_axis_name="c", subcore_axis_name="s",
                                    num_cores=NC, num_subcores=NS),
        scratch_types=[
            pltpu.VMEM((K,), jnp.int32),
            pltpu.VMEM((K,), jnp.int32),
            pltpu.VMEM((K,), jnp.int32),
            pltpu.VMEM((K, H2), F32),
            pltpu.VMEM((K, H2), F32),
            pltpu.VMEM((K,), jnp.int32),
            pltpu.VMEM((K,), jnp.int32),
            pltpu.VMEM((K,), jnp.int32),
            pltpu.VMEM((K, H2), F32),
            pltpu.VMEM((K, H2), F32),
            pltpu.VMEM_SHARED((NNP, H2), F32),
            pltpu.SemaphoreType.DMA,
            pltpu.SemaphoreType.DMA,
            pltpu.SemaphoreType.DMA,
            pltpu.SemaphoreType.DMA,
        ],
    )


def _mp_call(*args):
    return _mp_kernel()(*args)


def _mlp1_body(eps_ref, h_ref, a2_ref, w1_ref, b1_ref, w2_ref, b2_ref,
               z2_ref, ps_ref):
    aggr = jnp.concatenate([a2_ref[0], a2_ref[1]], axis=1)
    z = (1.0 + eps_ref[0, 0]) * h_ref[...] + aggr
    hid = jnp.maximum(
        jnp.dot(z, w1_ref[...], preferred_element_type=F32) + b1_ref[...], 0.0)
    z2 = jnp.dot(hid, w2_ref[...], preferred_element_type=F32) + b2_ref[...]
    z2_ref[...] = z2
    ps_ref[0, 0:1, :] = jnp.sum(z2, axis=0, keepdims=True)
    ps_ref[0, 1:2, :] = jnp.sum(z2 * z2, axis=0, keepdims=True)


def _mlp1_call(eps, h, a3, w1, b1, w2, b2):
    return pl.pallas_call(
        _mlp1_body,
        grid=(NB,),
        in_specs=[
            pl.BlockSpec((1, 1), lambda i: (0, 0)),
            pl.BlockSpec((BLK, HID), lambda i: (i, 0)),
            pl.BlockSpec((2, BLK, H2), lambda i: (0, i, 0)),
            pl.BlockSpec((HID, HID), lambda i: (0, 0)),
            pl.BlockSpec((1, HID), lambda i: (0, 0)),
            pl.BlockSpec((HID, HID), lambda i: (0, 0)),
            pl.BlockSpec((1, HID), lambda i: (0, 0)),
        ],
        out_specs=[
            pl.BlockSpec((BLK, HID), lambda i: (i, 0)),
            pl.BlockSpec((1, 2, HID), lambda i: (i, 0, 0)),
        ],
        out_shape=[
            jax.ShapeDtypeStruct((NN, HID), F32),
            jax.ShapeDtypeStruct((NB, 2, HID), F32),
        ],
    )(eps, h, a3, w1, b1, w2, b2)


def _mlp2_body(h_ref, z2_ref, ps_ref, g_ref, b_ref, ho_ref, hc_ref):
    ps = ps_ref[...]
    s1 = jnp.sum(ps[:, 0:1, :], axis=0)
    s2 = jnp.sum(ps[:, 1:2, :], axis=0)
    mean = s1 / NN
    var = s2 / NN - mean * mean
    z = (z2_ref[...] - mean) * lax.rsqrt(var + 1e-5) * g_ref[...] + b_ref[...]
    z = jnp.maximum(z, 0.0)
    hn = h_ref[...] + z
    ho_ref[...] = hn
    hc_ref[0] = hn[:, :H2]
    hc_ref[1] = hn[:, H2:]


def _mlp2_call(h, z2, ps, bn_g, bn_b):
    return pl.pallas_call(
        _mlp2_body,
        grid=(NB,),
        in_specs=[
            pl.BlockSpec((BLK, HID), lambda i: (i, 0)),
            pl.BlockSpec((BLK, HID), lambda i: (i, 0)),
            pl.BlockSpec((NB, 2, HID), lambda i: (0, 0, 0)),
            pl.BlockSpec((1, HID), lambda i: (0, 0)),
            pl.BlockSpec((1, HID), lambda i: (0, 0)),
        ],
        out_specs=[
            pl.BlockSpec((BLK, HID), lambda i: (i, 0)),
            pl.BlockSpec((2, BLK, H2), lambda i: (0, i, 0)),
        ],
        out_shape=[
            jax.ShapeDtypeStruct((NN, HID), F32),
            jax.ShapeDtypeStruct((2, NN, H2), F32),
        ],
    )(h, z2, ps, bn_g, bn_b)


def _readout_body(h_ref, bt_ref, w1_ref, b1_ref, w2_ref, b2_ref, out_ref):
    h = h_ref[...]
    g1 = jnp.maximum(
        jnp.dot(h, w1_ref[...], preferred_element_type=F32) + b1_ref[...], 0.0)
    gate = jnp.dot(g1, w2_ref[...], preferred_element_type=F32) + b2_ref[...]
    oh = bt_ref[...] == lax.broadcasted_iota(jnp.int32, (NN, NG), 1)
    ohf = oh.astype(F32)
    gmax = jnp.max(jnp.where(oh, gate, -3e38), axis=0, keepdims=True)
    gm_n = jnp.sum(jnp.where(oh, gmax, 0.0), axis=1, keepdims=True)
    gexp = jnp.exp(gate - gm_n)
    denom = jnp.sum(ohf * gexp, axis=0, keepdims=True)
    den_n = jnp.sum(jnp.where(oh, denom, 0.0), axis=1, keepdims=True)
    alpha = gexp / den_n
    out_ref[...] = lax.dot_general(ohf, alpha * h, (((0,), (0,)), ((), ())),
                                   preferred_element_type=F32)


def _readout_call(h, bt, w1, b1, w2, b2):
    return pl.pallas_call(
        _readout_body,
        out_shape=jax.ShapeDtypeStruct((NG, HID), F32),
    )(h, bt, w1, b1, w2, b2)


def kernel(params, x, edge_index, edge_attr, batch):
    p = params
    estack = jnp.stack(p["edge_embs"])
    wstack = jnp.stack([lp["linW"] for lp in p["layers"]])
    bstack = jnp.stack([lp["linb"] for lp in p["layers"]])[:, None, :]
    tcat = _tables_call(estack, wstack, bstack)

    ntab = jnp.stack(p["node_embs"])
    h, hcat = _embed_call(x.astype(jnp.int32), ntab,
                          p["ln_g"][None], p["ln_b"][None])

    src = edge_index[0].astype(jnp.int32)
    dst = edge_index[1].astype(jnp.int32)
    ea = edge_attr.astype(jnp.int32)
    eidx = ea[:, 0] * (EVOC * EVOC) + ea[:, 1] * EVOC + ea[:, 2]
    src2 = jnp.concatenate([src, src + NN])
    eidx2 = jnp.concatenate([eidx, eidx + TBL])
    zeros = jnp.zeros((RPS, H2), F32)

    for l in range(NL):
        lp = p["layers"][l]
        aggr2 = _mp_call(hcat.reshape(2 * NN, H2), tcat[l], src2, eidx2,
                         dst, zeros)
        a3 = aggr2.reshape(2, NNP, H2)[:, :NN, :]
        z2, ps = _mlp1_call(lp["eps"].reshape(1, 1).astype(F32), h, a3,
                            lp["W1"], lp["b1"][None], lp["W2"], lp["b2"][None])
        h, hcat = _mlp2_call(h, z2, ps, lp["bn_g"][None], lp["bn_b"][None])

    graph_emb = _readout_call(h, batch.astype(jnp.int32)[:, None],
                              p["gW1"], p["gb1"][None], p["gW2"],
                              p["gb2"].reshape(1, 1))
    return graph_emb, h
